# baseline (speedup 1.0000x reference)
import jax
import jax.numpy as jnp
from jax.experimental import pallas as pl

_VERTEX_IDS = (9120, 9929, 9448, 616, 6,
               5770, 5780, 8846, 8463, 8474, 8635,
               5361, 4933, 5058, 5169, 5286,
               8079, 7669, 7794, 7905, 8022)

_B = 1024
_V = 10475
_J = 55
_E = len(_VERTEX_IDS)


def _body(jt_ref, *refs):
    stripe_refs = refs[: 3 * _E]
    out_ref = refs[3 * _E]
    out_ref[:, : _J, :] = jt_ref[...]
    for j, idx in enumerate(_VERTEX_IDS):
        for c in range(3):
            out_ref[c, _J + j, :] = stripe_refs[3 * j + c][0, idx % 8, :]


def kernel(vertices, joints):
    vt = vertices.transpose(2, 1, 0)
    jt = joints.transpose(2, 1, 0)

    def stripe_spec(idx, c):
        blk = idx // 8
        return pl.BlockSpec((1, 8, _B), lambda i, blk=blk, c=c: (c, blk, 0))

    out_t = pl.pallas_call(
        _body,
        grid=(1,),
        out_shape=jax.ShapeDtypeStruct((3, _J + _E, _B), jnp.float32),
        in_specs=[pl.BlockSpec((3, _J, _B), lambda i: (0, 0, 0))]
        + [stripe_spec(idx, c) for idx in _VERTEX_IDS for c in range(3)],
        out_specs=pl.BlockSpec((3, _J + _E, _B), lambda i: (0, 0, 0)),
    )(jt, *([vt] * (3 * _E)))
    return out_t.transpose(2, 1, 0)


# speedup vs baseline: 1.0567x; 1.0567x over previous
import jax
import jax.numpy as jnp
from jax.experimental import pallas as pl

_VERTEX_IDS = (9120, 9929, 9448, 616, 6,
               5770, 5780, 8846, 8463, 8474, 8635,
               5361, 4933, 5058, 5169, 5286,
               8079, 7669, 7794, 7905, 8022)

_B = 1024
_V = 10475
_J = 55
_E = len(_VERTEX_IDS)


def _body(jt_ref, *refs):
    stripe_refs = refs[:_E]
    out_ref = refs[_E]
    out_ref[:, : _J, :] = jt_ref[...]
    for j, idx in enumerate(_VERTEX_IDS):
        out_ref[:, _J + j, :] = stripe_refs[j][:, idx % 8, :]


def kernel(vertices, joints):
    vt = vertices.transpose(2, 1, 0)
    jt = joints.transpose(2, 1, 0)

    def stripe_spec(idx):
        blk = idx // 8
        return pl.BlockSpec((3, 8, _B), lambda i, blk=blk: (0, blk, 0))

    out_t = pl.pallas_call(
        _body,
        grid=(1,),
        out_shape=jax.ShapeDtypeStruct((3, _J + _E, _B), jnp.float32),
        in_specs=[pl.BlockSpec((3, _J, _B), lambda i: (0, 0, 0))]
        + [stripe_spec(idx) for idx in _VERTEX_IDS],
        out_specs=pl.BlockSpec((3, _J + _E, _B), lambda i: (0, 0, 0)),
    )(jt, *([vt] * _E))
    return out_t.transpose(2, 1, 0)
